# baseline (speedup 1.0000x reference)
import functools

import jax
import jax.numpy as jnp
from jax.experimental import pallas as pl

F32 = jnp.float32
BF16 = jnp.bfloat16
_B = 64
_N = 26
_NN = _N * _N
_NF = 96
_C0 = 2 * _NF
_GD = _NF // 2
_NK = 5
_EPS = 1e-5
_PREC = jax.lax.Precision.DEFAULT

_INTERPRET = False


def _dot(a, b):
    return jax.lax.dot_general(a.astype(BF16), b.astype(BF16),
                               (((a.ndim - 1,), (0,)), ((), ())),
                               precision=_PREC, preferred_element_type=F32)


def _lrelu(x):
    return jnp.where(x >= 0, x, 0.01 * x)


def _accum(sum_ref, h):
    s = jnp.concatenate(
        [jnp.sum(h, axis=0, keepdims=True), jnp.sum(h * h, axis=0, keepdims=True)],
        axis=0)
    @pl.when(pl.program_id(0) == 0)
    def _():
        sum_ref[...] = s

    @pl.when(pl.program_id(0) != 0)
    def _():
        sum_ref[...] += s


def _bn_coeffs(sums, count, g, be):
    mean = sums[0] / count
    var = sums[1] / count - mean * mean
    scale = g * jax.lax.rsqrt(var + _EPS)
    shift = be - mean * scale
    return scale.reshape(1, -1), shift.reshape(1, -1)


def _pass_a_body(x, w, b, h_ref, sum_ref, ep, d):
    diff = jnp.abs(x[:, :, None, :] - x[:, None, :, :])
    x0 = diff.reshape(ep * _NN, d)
    h = _dot(x0, w) + b
    h_ref[...] = h.astype(h_ref.dtype)
    _accum(sum_ref, h)


def _pass_a_kernel(x_ref, w_ref, b_ref, h_ref, sum_ref, *, ep, d):
    _pass_a_body(x_ref[...], w_ref[...], b_ref[...], h_ref, sum_ref, ep, d)


def _pass_a_cat_kernel(x_ref, gp_ref, gs_ref, gb_ref, w_ref, b_ref,
                       h_ref, sum_ref, xcat_ref, *, ep, d):
    xn = _lrelu(gp_ref[...] * gs_ref[...] + gb_ref[...])
    x = jnp.concatenate([x_ref[...], xn], axis=-1)
    xcat_ref[...] = x
    _pass_a_body(x, w_ref[...], b_ref[...], h_ref, sum_ref, ep, d)


def _run_pass_a(x, w, b, ep):
    d = x.shape[-1]
    grid = _B // ep
    return pl.pallas_call(
        functools.partial(_pass_a_kernel, ep=ep, d=d),
        grid=(grid,),
        in_specs=[
            pl.BlockSpec((ep, _N, d), lambda g: (g, 0, 0)),
            pl.BlockSpec((d, _C0), lambda g: (0, 0)),
            pl.BlockSpec((1, _C0), lambda g: (0, 0)),
        ],
        out_specs=[
            pl.BlockSpec((ep * _NN, _C0), lambda g: (g, 0)),
            pl.BlockSpec((2, _C0), lambda g: (0, 0)),
        ],
        out_shape=[
            jax.ShapeDtypeStruct((_B * _NN, _C0), BF16),
            jax.ShapeDtypeStruct((2, _C0), F32),
        ],
        interpret=_INTERPRET,
    )(x, w, b.reshape(1, -1))


def _run_pass_a_cat(x_old, gpre, gs, gb, w, b, ep):
    d_old = x_old.shape[-1]
    d = d_old + _GD
    grid = _B // ep
    h, sums, xcat = pl.pallas_call(
        functools.partial(_pass_a_cat_kernel, ep=ep, d=d),
        grid=(grid,),
        in_specs=[
            pl.BlockSpec((ep, _N, d_old), lambda g: (g, 0, 0)),
            pl.BlockSpec((ep, _N, _GD), lambda g: (g, 0, 0)),
            pl.BlockSpec((1, 1, _GD), lambda g: (0, 0, 0)),
            pl.BlockSpec((1, 1, _GD), lambda g: (0, 0, 0)),
            pl.BlockSpec((d, _C0), lambda g: (0, 0)),
            pl.BlockSpec((1, _C0), lambda g: (0, 0)),
        ],
        out_specs=[
            pl.BlockSpec((ep * _NN, _C0), lambda g: (g, 0)),
            pl.BlockSpec((2, _C0), lambda g: (0, 0)),
            pl.BlockSpec((ep, _N, d), lambda g: (g, 0, 0)),
        ],
        out_shape=[
            jax.ShapeDtypeStruct((_B * _NN, _C0), BF16),
            jax.ShapeDtypeStruct((2, _C0), F32),
            jax.ShapeDtypeStruct((_B, _N, d), F32),
        ],
        interpret=_INTERPRET,
    )(x_old, gpre, gs.reshape(1, 1, -1), gb.reshape(1, 1, -1), w, b.reshape(1, -1))
    return h, sums, xcat


def _pass_p_kernel(h_ref, sc_ref, sh_ref, w_ref, b_ref, out_ref, sum_ref):
    x = _lrelu(h_ref[...].astype(F32) * sc_ref[...] + sh_ref[...])
    h = _dot(x, w_ref[...]) + b_ref[...]
    out_ref[...] = h.astype(out_ref.dtype)
    _accum(sum_ref, h)


def _run_pass_p(h_prev, sc, sh, w, b, ep):
    cin = h_prev.shape[-1]
    cout = w.shape[-1]
    grid = _B // ep
    return pl.pallas_call(
        _pass_p_kernel,
        grid=(grid,),
        in_specs=[
            pl.BlockSpec((ep * _NN, cin), lambda g: (g, 0)),
            pl.BlockSpec((1, cin), lambda g: (0, 0)),
            pl.BlockSpec((1, cin), lambda g: (0, 0)),
            pl.BlockSpec((cin, cout), lambda g: (0, 0)),
            pl.BlockSpec((1, cout), lambda g: (0, 0)),
        ],
        out_specs=[
            pl.BlockSpec((ep * _NN, cout), lambda g: (g, 0)),
            pl.BlockSpec((2, cout), lambda g: (0, 0)),
        ],
        out_shape=[
            jax.ShapeDtypeStruct((_B * _NN, cout), BF16),
            jax.ShapeDtypeStruct((2, cout), F32),
        ],
        interpret=_INTERPRET,
    )(h_prev, sc, sh, w, b.reshape(1, -1))


def _pass_p3_kernel(h_ref, sc_ref, sh_ref, w_ref, b_ref, out_ref):
    x = _lrelu(h_ref[...].astype(F32) * sc_ref[...] + sh_ref[...])
    out_ref[...] = _dot(x, w_ref[...]) + b_ref[...]


def _run_pass_p3(h_prev, sc, sh, w, b, ep):
    cin = h_prev.shape[-1]
    grid = _B // ep
    return pl.pallas_call(
        _pass_p3_kernel,
        grid=(grid,),
        in_specs=[
            pl.BlockSpec((ep * _NN, cin), lambda g: (g, 0)),
            pl.BlockSpec((1, cin), lambda g: (0, 0)),
            pl.BlockSpec((1, cin), lambda g: (0, 0)),
            pl.BlockSpec((cin, 1), lambda g: (0, 0)),
            pl.BlockSpec((1, 1), lambda g: (0, 0)),
        ],
        out_specs=pl.BlockSpec((ep * _NN, 1), lambda g: (g, 0)),
        out_shape=jax.ShapeDtypeStruct((_B * _NN, 1), F32),
        interpret=_INTERPRET,
    )(h_prev, sc, sh, w, b.reshape(1, 1))


def _pass_g_kernel(lg_ref, x_ref, w_ref, b_ref, out_ref, sum_ref, *, ep, d):
    lg = lg_ref[...]
    row = jax.lax.broadcasted_iota(jnp.int32, (_N, _N), 0)
    col = jax.lax.broadcasted_iota(jnp.int32, (_N, _N), 1)
    eye = (row == col).astype(F32)
    lg = lg - 1e8 * eye[None]
    m = jnp.max(lg, axis=-1, keepdims=True)
    e = jnp.exp(lg - m)
    a = e / jnp.sum(e, axis=-1, keepdims=True)
    x = x_ref[...]
    w = w_ref[...]
    b = b_ref[...]
    for i in range(ep):
        ax = _dot(a[i], x[i])
        cat = jnp.concatenate([x[i], ax], axis=-1)
        h = _dot(cat, w) + b
        out_ref[i] = h
        s = jnp.concatenate([jnp.sum(h, axis=0, keepdims=True),
                             jnp.sum(h * h, axis=0, keepdims=True)], axis=0)
        if i == 0:
            @pl.when(pl.program_id(0) == 0)
            def _():
                sum_ref[...] = s

            @pl.when(pl.program_id(0) != 0)
            def _():
                sum_ref[...] += s
        else:
            sum_ref[...] += s


def _run_pass_g(logits, x, w, b, ep):
    d = x.shape[-1]
    grid = _B // ep
    return pl.pallas_call(
        functools.partial(_pass_g_kernel, ep=ep, d=d),
        grid=(grid,),
        in_specs=[
            pl.BlockSpec((ep, _N, _N), lambda g: (g, 0, 0)),
            pl.BlockSpec((ep, _N, d), lambda g: (g, 0, 0)),
            pl.BlockSpec((2 * d, _GD), lambda g: (0, 0)),
            pl.BlockSpec((1, _GD), lambda g: (0, 0)),
        ],
        out_specs=[
            pl.BlockSpec((ep, _N, _GD), lambda g: (g, 0, 0)),
            pl.BlockSpec((2, _GD), lambda g: (0, 0)),
        ],
        out_shape=[
            jax.ShapeDtypeStruct((_B, _N, _GD), F32),
            jax.ShapeDtypeStruct((2, _GD), F32),
        ],
        interpret=_INTERPRET,
    )(logits, x, w, b.reshape(1, -1))


def _pass_f_kernel(h_ref, sc_ref, sh_ref, w4_ref, b4_ref, x_ref, wg_ref, bg_ref,
                   sig_ref, log_ref):
    x4 = _lrelu(h_ref[...].astype(F32) * sc_ref[...] + sh_ref[...])
    h4 = jnp.sum(x4 * w4_ref[...], axis=-1) + b4_ref[0, 0]
    col = jax.lax.broadcasted_iota(jnp.int32, (_B, _N), 1)
    h4 = h4 - 1e8 * (col == 0).astype(F32)
    m = jnp.max(h4, axis=-1, keepdims=True)
    e = jnp.exp(h4 - m)
    a = e / jnp.sum(e, axis=-1, keepdims=True)
    x = x_ref[...]
    ax = jnp.sum(a[:, :, None] * x, axis=1)
    cat = jnp.concatenate([x[:, 0, :], ax], axis=-1)
    logits = _dot(cat, wg_ref[...]) + bg_ref[...]
    log_ref[...] = logits
    sig_ref[...] = 1.0 / (1.0 + jnp.exp(-logits))


def _run_pass_f(h3_row0, sc, sh, w4, b4, x, wg, bg):
    d = x.shape[-1]
    return pl.pallas_call(
        _pass_f_kernel,
        grid=(1,),
        in_specs=[
            pl.BlockSpec((_B, _N, _NF), lambda g: (0, 0, 0)),
            pl.BlockSpec((1, 1, _NF), lambda g: (0, 0, 0)),
            pl.BlockSpec((1, 1, _NF), lambda g: (0, 0, 0)),
            pl.BlockSpec((1, 1, _NF), lambda g: (0, 0, 0)),
            pl.BlockSpec((1, 1), lambda g: (0, 0)),
            pl.BlockSpec((_B, _N, d), lambda g: (0, 0, 0)),
            pl.BlockSpec((2 * d, _NK), lambda g: (0, 0)),
            pl.BlockSpec((1, _NK), lambda g: (0, 0)),
        ],
        out_specs=[
            pl.BlockSpec((_B, _NK), lambda g: (0, 0)),
            pl.BlockSpec((_B, _NK), lambda g: (0, 0)),
        ],
        out_shape=[
            jax.ShapeDtypeStruct((_B, _NK), F32),
            jax.ShapeDtypeStruct((_B, _NK), F32),
        ],
        interpret=_INTERPRET,
    )(h3_row0, sc.reshape(1, 1, -1), sh.reshape(1, 1, -1), w4.reshape(1, 1, -1),
      b4.reshape(1, 1), x, wg, bg.reshape(1, -1))


_M_PAIR = float(_B * _NN)
_M_NODE = float(_B * _N)
_EP = 4
_EP_G = 16


def _wcompute_mlp(h0, sums0, p):
    sc0, sh0 = _bn_coeffs(sums0, _M_PAIR, p['g0'], p['be0'])
    h1, sums1 = _run_pass_p(h0, sc0, sh0, p['w1'], p['b1'], _EP)
    sc1, sh1 = _bn_coeffs(sums1, _M_PAIR, p['g1'], p['be1'])
    h2, sums2 = _run_pass_p(h1, sc1, sh1, p['w2'], p['b2'], _EP)
    sc2, sh2 = _bn_coeffs(sums2, _M_PAIR, p['g2'], p['be2'])
    h3, sums3 = _run_pass_p(h2, sc2, sh2, p['w3'], p['b3'], _EP)
    sc3, sh3 = _bn_coeffs(sums3, _M_PAIR, p['g3'], p['be3'])
    return h3, sc3, sh3


def kernel(z, zi_s, labels_yi, zero_pad, params):
    labels = jnp.concatenate([zero_pad[None], labels_yi], axis=0)
    feats = jnp.concatenate([z[None], zi_s], axis=0)
    nodes = jnp.concatenate([feats, labels], axis=2)
    x0 = jnp.transpose(nodes, (1, 0, 2))

    p = params['wc0']
    h0, sums0 = _run_pass_a(x0, p['w0'], p['b0'], _EP)
    h3, sc3, sh3 = _wcompute_mlp(h0, sums0, p)
    lg = _run_pass_p3(h3, sc3, sh3, p['w4'], p['b4'], _EP)
    lg = lg.reshape(_B, _N, _N)
    gp = params['gc0']
    gpre0, gsum0 = _run_pass_g(lg, x0, gp['w'], gp['b'], _EP_G)
    gs0, gb0 = _bn_coeffs(gsum0, _M_NODE, gp['g'], gp['be'])

    p = params['wc1']
    h0, sums0, x1 = _run_pass_a_cat(x0, gpre0, gs0, gb0, p['w0'], p['b0'], _EP)
    h3, sc3, sh3 = _wcompute_mlp(h0, sums0, p)
    lg = _run_pass_p3(h3, sc3, sh3, p['w4'], p['b4'], _EP)
    lg = lg.reshape(_B, _N, _N)
    gp = params['gc1']
    gpre1, gsum1 = _run_pass_g(lg, x1, gp['w'], gp['b'], _EP_G)
    gs1, gb1 = _bn_coeffs(gsum1, _M_NODE, gp['g'], gp['be'])

    p = params['wcl']
    h0, sums0, x2 = _run_pass_a_cat(x1, gpre1, gs1, gb1, p['w0'], p['b0'], _EP)
    h3, sc3, sh3 = _wcompute_mlp(h0, sums0, p)
    h3_row0 = h3.reshape(_B, _NN, _NF)[:, :_N, :]
    gp = params['gcl']
    sig, logits = _run_pass_f(h3_row0, sc3, sh3, p['w4'], p['b4'], x2,
                              gp['w'], gp['b'])
    return (sig, logits)


# speedup vs baseline: 1.4446x; 1.4446x over previous
import functools

import jax
import jax.numpy as jnp
from jax.experimental import pallas as pl
from jax.experimental.pallas import tpu as pltpu

F32 = jnp.float32
BF16 = jnp.bfloat16
_B = 64
_N = 26
_NN = _N * _N
_R = _B * _NN
_NF = 96
_C0 = 2 * _NF
_GD = _NF // 2
_NK = 5
_EPS = 1e-5
_PREC = jax.lax.Precision.DEFAULT

_CHE = 8
_NCH = _B // _CHE
_CH = _CHE * _NN

_INTERPRET = False


def _dot(a, b):
    return jax.lax.dot_general(a, b, (((a.ndim - 1,), (0,)), ((), ())),
                               precision=_PREC, preferred_element_type=F32)


def _lrelu(x):
    return jnp.where(x >= 0, x, 0.01 * x)


def _coeffs(s1, s2, cnt, g, be):
    mean = s1 / cnt
    var = s2 / cnt - mean * mean
    sc = g * jax.lax.rsqrt(var + _EPS)
    sh = be - mean * sc
    return sc, sh


def _sums(h):
    return jnp.sum(h, axis=0, keepdims=True), jnp.sum(h * h, axis=0, keepdims=True)


def _mega_kernel(*refs, d, cat, final):
    if cat:
        (x_ref, gp_ref, gsum_ref, gg_ref, gbe_ref, w0, b0, g0, be0, w1, b1,
         g1, be1, w2, b2, g2, be2, w3, b3, g3, be3, w4, b4) = refs[:23]
        refs = refs[23:]
        if final:
            h3r_ref, sc3_ref, sh3_ref, xc_ref, ha_ref, hb_ref = refs
        else:
            lg_ref, xc_ref, ha_ref, hb_ref = refs
    else:
        (x_ref, w0, b0, g0, be0, w1, b1, g1, be1, w2, b2, g2, be2, w3, b3,
         g3, be3, w4, b4) = refs[:19]
        refs = refs[19:]
        lg_ref, ha_ref, hb_ref = refs
        xc_ref = x_ref

    if cat:
        gs, gb = _coeffs(gsum_ref[0:1], gsum_ref[1:2], float(_B * _N),
                         gg_ref[...], gbe_ref[...])
        xn = _lrelu(gp_ref[...] * gs[None] + gb[None])
        xc_ref[...] = jnp.concatenate([x_ref[...], xn], axis=-1)

    zc0 = jnp.zeros((1, _C0), F32)
    zc1 = jnp.zeros((1, _NF), F32)

    def sweep0(c, carry):
        s1, s2 = carry
        xb = xc_ref[pl.ds(c * _CHE, _CHE)]
        diff = jnp.abs(xb[:, :, None, :] - xb[:, None, :, :])
        h = _dot(diff.reshape(_CH, d), w0[...]) + b0[...]
        ha_ref[pl.ds(c * _CH, _CH), :] = h.astype(BF16)
        a1, a2 = _sums(h)
        return s1 + a1, s2 + a2

    s1, s2 = jax.lax.fori_loop(0, _NCH, sweep0, (zc0, zc0))
    sc, sh = _coeffs(s1, s2, float(_R), g0[...], be0[...])

    def make_sweep(src, s_cin, dst, w, b, sc, sh):
        def body(c, carry):
            s1, s2 = carry
            hp = src[pl.ds(c * _CH, _CH), :s_cin].astype(F32)
            xk = _lrelu(hp * sc + sh)
            h = _dot(xk, w[...]) + b[...]
            dst[pl.ds(c * _CH, _CH), :h.shape[1]] = h.astype(BF16)
            a1, a2 = _sums(h)
            return s1 + a1, s2 + a2
        return body

    s1, s2 = jax.lax.fori_loop(0, _NCH,
                               make_sweep(ha_ref, _C0, hb_ref, w1, b1, sc, sh),
                               (zc0, zc0))
    sc, sh = _coeffs(s1, s2, float(_R), g1[...], be1[...])

    s1, s2 = jax.lax.fori_loop(0, _NCH,
                               make_sweep(hb_ref, _C0, ha_ref, w2, b2, sc, sh),
                               (zc1, zc1))
    sc, sh = _coeffs(s1, s2, float(_R), g2[...], be2[...])

    s1, s2 = jax.lax.fori_loop(0, _NCH,
                               make_sweep(ha_ref, _NF, hb_ref, w3, b3, sc, sh),
                               (zc1, zc1))
    sc3, sh3 = _coeffs(s1, s2, float(_R), g3[...], be3[...])

    if final:
        def extract(c, _):
            blk = hb_ref[pl.ds(c * _CH, _CH), :_NF]
            for k in range(_CHE):
                h3r_ref[pl.ds(c * _CHE + k, 1)] = blk[k * _NN:k * _NN + _N][None]
            return 0
        jax.lax.fori_loop(0, _NCH, extract, 0)
        sc3_ref[...] = sc3
        sh3_ref[...] = sh3
    else:
        def sweep4(c, _):
            hp = hb_ref[pl.ds(c * _CH, _CH), :_NF].astype(F32)
            x4 = _lrelu(hp * sc3 + sh3)
            h4 = _dot(x4, w4[...]) + b4[...]
            lg_ref[pl.ds(c, 1), :] = h4.reshape(1, _CH)
            return 0
        jax.lax.fori_loop(0, _NCH, sweep4, 0)


def _full(shape):
    n = len(shape)
    return pl.BlockSpec(shape, lambda: (0,) * n)


def _run_mega(x, gpre, gsums, gg, gbe, p, final):
    d_old = x.shape[-1]
    cat = gpre is not None
    d = d_old + (_GD if cat else 0)

    wb = []
    for i in range(4):
        wb += [p['w%d' % i], p['b%d' % i].reshape(1, -1),
               p['g%d' % i].reshape(1, -1), p['be%d' % i].reshape(1, -1)]
    wb += [p['w4'], p['b4'].reshape(1, 1)]
    args = [x] + ([gpre, gsums, gg.reshape(1, -1), gbe.reshape(1, -1)] if cat
                  else []) + wb
    in_specs = [_full(a.shape) for a in args]

    out_specs, out_shape = [], []
    if final:
        out_specs += [_full((_B, _N, _NF)), _full((1, _NF)), _full((1, _NF))]
        out_shape += [jax.ShapeDtypeStruct((_B, _N, _NF), BF16),
                      jax.ShapeDtypeStruct((1, _NF), F32),
                      jax.ShapeDtypeStruct((1, _NF), F32)]
    else:
        out_specs += [_full((_NCH, _CH))]
        out_shape += [jax.ShapeDtypeStruct((_NCH, _CH), F32)]
    if cat:
        out_specs += [_full((_B, _N, d))]
        out_shape += [jax.ShapeDtypeStruct((_B, _N, d), F32)]

    return pl.pallas_call(
        functools.partial(_mega_kernel, d=d, cat=cat, final=final),
        in_specs=in_specs,
        out_specs=out_specs,
        out_shape=out_shape,
        scratch_shapes=[pltpu.VMEM((_R, _C0), BF16),
                        pltpu.VMEM((_R, _C0), BF16)],
        interpret=_INTERPRET,
    )(*args)


def _pass_g_kernel(lg_ref, x_ref, w_ref, b_ref, out_ref, sum_ref, *, ep, d):
    lg = lg_ref[...]
    row = jax.lax.broadcasted_iota(jnp.int32, (_N, _N), 0)
    col = jax.lax.broadcasted_iota(jnp.int32, (_N, _N), 1)
    eye = (row == col).astype(F32)
    lg = lg - 1e8 * eye[None]
    m = jnp.max(lg, axis=-1, keepdims=True)
    e = jnp.exp(lg - m)
    a = e / jnp.sum(e, axis=-1, keepdims=True)
    x = x_ref[...]
    w = w_ref[...]
    b = b_ref[...]
    for i in range(ep):
        ax = _dot(a[i], x[i])
        cat = jnp.concatenate([x[i], ax], axis=-1)
        h = _dot(cat, w) + b
        out_ref[i] = h
        s = jnp.concatenate([jnp.sum(h, axis=0, keepdims=True),
                             jnp.sum(h * h, axis=0, keepdims=True)], axis=0)
        if i == 0:
            @pl.when(pl.program_id(0) == 0)
            def _():
                sum_ref[...] = s

            @pl.when(pl.program_id(0) != 0)
            def _():
                sum_ref[...] += s
        else:
            sum_ref[...] += s


_EP_G = 16


def _run_pass_g(logits, x, w, b):
    d = x.shape[-1]
    grid = _B // _EP_G
    return pl.pallas_call(
        functools.partial(_pass_g_kernel, ep=_EP_G, d=d),
        grid=(grid,),
        in_specs=[
            pl.BlockSpec((_EP_G, _N, _N), lambda g: (g, 0, 0)),
            pl.BlockSpec((_EP_G, _N, d), lambda g: (g, 0, 0)),
            pl.BlockSpec((2 * d, _GD), lambda g: (0, 0)),
            pl.BlockSpec((1, _GD), lambda g: (0, 0)),
        ],
        out_specs=[
            pl.BlockSpec((_EP_G, _N, _GD), lambda g: (g, 0, 0)),
            pl.BlockSpec((2, _GD), lambda g: (0, 0)),
        ],
        out_shape=[
            jax.ShapeDtypeStruct((_B, _N, _GD), F32),
            jax.ShapeDtypeStruct((2, _GD), F32),
        ],
        interpret=_INTERPRET,
    )(logits, x, w, b.reshape(1, -1))


def _pass_f_kernel(h_ref, sc_ref, sh_ref, w4_ref, b4_ref, x_ref, wg_ref, bg_ref,
                   sig_ref, log_ref):
    x4 = _lrelu(h_ref[...].astype(F32) * sc_ref[...] + sh_ref[...])
    h4 = jnp.sum(x4 * w4_ref[...], axis=-1) + b4_ref[0, 0]
    col = jax.lax.broadcasted_iota(jnp.int32, (_B, _N), 1)
    h4 = h4 - 1e8 * (col == 0).astype(F32)
    m = jnp.max(h4, axis=-1, keepdims=True)
    e = jnp.exp(h4 - m)
    a = e / jnp.sum(e, axis=-1, keepdims=True)
    x = x_ref[...]
    ax = jnp.sum(a[:, :, None] * x, axis=1)
    cat = jnp.concatenate([x[:, 0, :], ax], axis=-1)
    logits = _dot(cat, wg_ref[...]) + bg_ref[...]
    log_ref[...] = logits
    sig_ref[...] = 1.0 / (1.0 + jnp.exp(-logits))


def _run_pass_f(h3_row0, sc, sh, w4, b4, x, wg, bg):
    d = x.shape[-1]
    return pl.pallas_call(
        _pass_f_kernel,
        in_specs=[
            _full((_B, _N, _NF)),
            _full((1, 1, _NF)),
            _full((1, 1, _NF)),
            _full((1, 1, _NF)),
            _full((1, 1)),
            _full((_B, _N, d)),
            _full((2 * d, _NK)),
            _full((1, _NK)),
        ],
        out_specs=[
            _full((_B, _NK)),
            _full((_B, _NK)),
        ],
        out_shape=[
            jax.ShapeDtypeStruct((_B, _NK), F32),
            jax.ShapeDtypeStruct((_B, _NK), F32),
        ],
        interpret=_INTERPRET,
    )(h3_row0, sc.reshape(1, 1, -1), sh.reshape(1, 1, -1), w4.reshape(1, 1, -1),
      b4.reshape(1, 1), x, wg, bg.reshape(1, -1))


def kernel(z, zi_s, labels_yi, zero_pad, params):
    labels = jnp.concatenate([zero_pad[None], labels_yi], axis=0)
    feats = jnp.concatenate([z[None], zi_s], axis=0)
    nodes = jnp.concatenate([feats, labels], axis=2)
    x0 = jnp.transpose(nodes, (1, 0, 2))

    lg, = _run_mega(x0, None, None, None, None, params['wc0'], False)
    gp = params['gc0']
    gpre0, gsum0 = _run_pass_g(lg.reshape(_B, _N, _N), x0, gp['w'], gp['b'])

    lg, x1 = _run_mega(x0, gpre0, gsum0, gp['g'], gp['be'], params['wc1'], False)
    gp = params['gc1']
    gpre1, gsum1 = _run_pass_g(lg.reshape(_B, _N, _N), x1, gp['w'], gp['b'])

    p = params['wcl']
    h3r, sc3, sh3, x2 = _run_mega(x1, gpre1, gsum1, gp['g'], gp['be'], p, True)
    gp = params['gcl']
    sig, logits = _run_pass_f(h3r, sc3, sh3, p['w4'], p['b4'], x2,
                              gp['w'], gp['b'])
    return (sig, logits)
